# baseline (speedup 1.0000x reference)
import functools
import jax
import jax.numpy as jnp
from jax import lax
from jax.experimental import pallas as pl
from jax.experimental.pallas import tpu as pltpu
from jax.experimental.pallas import tpu_sc as plsc

N = 16384
F = 512
H = 128
S = 8
BLK = 1024
NBLK = N // BLK

NC = 2
NS = 16
NW = NC * NS
RPW = N // NW
CH = 128
NCH = RPW // CH
NSL = F // 16

_mesh = plsc.VectorSubcoreMesh(core_axis_name="c", subcore_axis_name="s")


@functools.partial(
    pl.kernel,
    mesh=_mesh,
    out_type=jax.ShapeDtypeStruct((NW, S, F), jnp.float32),
    scratch_types=[
        pltpu.VMEM((16,), jnp.int32),
        pltpu.VMEM((CH, F), jnp.float32),
        pltpu.VMEM((S, F), jnp.float32),
        pltpu.SemaphoreType.DMA,
    ],
)
def _sc_segsum(b_hbm, x_hbm, out_hbm, bnd_v, xbuf, acc_v, sem0):
    cid = lax.axis_index("c")
    sid = lax.axis_index("s")
    wid = sid * NC + cid
    base = wid * RPW

    for i in range(S):
        for j in range(NSL):
            acc_v[i, pl.ds(j * 16, 16)] = jnp.zeros((16,), jnp.float32)

    pltpu.sync_copy(b_hbm.at[wid], bnd_v)
    bnd = bnd_v[...]
    b_at = [bnd[k] for k in range(S + 1)]

    for c in range(NCH):
        pltpu.sync_copy(x_hbm.at[pl.ds(base + c * CH, CH)], xbuf)
        for s in range(S):
            a = jnp.clip(b_at[s] - c * CH, 0, CH)
            b = jnp.clip(b_at[s + 1] - c * CH, 0, CH)

            @pl.when(b > a)
            def _(a=a, b=b, s=s, c=c):
                def row_add(r, regs):
                    return tuple(
                        regs[j] + xbuf[r, pl.ds(j * 16, 16)]
                        for j in range(NSL))

                regs = lax.fori_loop(
                    a, b, row_add,
                    tuple(jnp.zeros((16,), jnp.float32)
                          for _ in range(NSL)))
                for j in range(NSL):
                    acc_v[s, pl.ds(j * 16, 16)] = (
                        acc_v[s, pl.ds(j * 16, 16)] + regs[j])

    pltpu.sync_copy(acc_v, out_hbm.at[wid])


def _apply_body(b2_ref, b3_ref, part_ref, W0_ref, b0_ref, W1_ref, b1_ref,
                x_hbm, out_hbm, xbuf_ref, rsem, wsem):
    read_copies = []
    for k in range(NBLK):
        c = pltpu.make_async_copy(
            x_hbm.at[pl.ds(k * BLK, BLK), :],
            xbuf_ref.at[pl.ds(k * BLK, BLK), :],
            rsem.at[k])
        c.start()
        read_copies.append(c)

    b2 = b2_ref[...]
    cnt = jnp.concatenate(
        [jnp.sum((b2 == s).astype(jnp.float32))[None] for s in range(S)])
    acc = jnp.sum(part_ref[...], axis=0)
    mean = acc / jnp.maximum(cnt, 1.0)[:, None]
    h = jnp.maximum(
        lax.dot_general(mean, W0_ref[...], (((1,), (0,)), ((), ())),
                        preferred_element_type=jnp.float32) + b0_ref[...],
        0.0)
    z = lax.dot_general(h, W1_ref[...], (((1,), (0,)), ((), ())),
                        preferred_element_type=jnp.float32) + b1_ref[...]
    gate = 1.0 / (1.0 + jnp.exp(-z))

    for k in range(NBLK):
        read_copies[k].wait()
        ids = b3_ref[k, 0, :]
        oh = (ids[:, None] == lax.broadcasted_iota(jnp.int32, (BLK, S), 1)
              ).astype(jnp.float32)
        y = lax.dot_general(oh, gate, (((1,), (0,)), ((), ())),
                            preferred_element_type=jnp.float32)
        xbuf_ref[pl.ds(k * BLK, BLK), :] *= y
        pltpu.make_async_copy(
            xbuf_ref.at[pl.ds(k * BLK, BLK), :],
            out_hbm.at[pl.ds(k * BLK, BLK), :],
            wsem.at[k]).start()

    for k in range(NBLK):
        pltpu.make_async_copy(
            xbuf_ref.at[pl.ds(k * BLK, BLK), :],
            out_hbm.at[pl.ds(k * BLK, BLK), :],
            wsem.at[k]).wait()


def kernel(x, batch, W0, b0, W1, b1):
    batch32 = batch.astype(jnp.int32)
    b3 = batch32.reshape(NBLK, 1, BLK)
    b2 = batch32.reshape(128, 128)
    starts = jnp.searchsorted(
        batch32, jnp.arange(S + 1, dtype=jnp.int32)).astype(jnp.int32)
    w = jnp.arange(NW, dtype=jnp.int32)[:, None]
    bnd = jnp.clip(starts[None, :], w * RPW, (w + 1) * RPW) - w * RPW
    bnd = jnp.pad(bnd, ((0, 0), (0, 16 - (S + 1))))

    part = _sc_segsum(bnd, x)

    out = pl.pallas_call(
        _apply_body,
        in_specs=[
            pl.BlockSpec(memory_space=pltpu.MemorySpace.VMEM),
            pl.BlockSpec(memory_space=pltpu.MemorySpace.VMEM),
            pl.BlockSpec(memory_space=pltpu.MemorySpace.VMEM),
            pl.BlockSpec(memory_space=pltpu.MemorySpace.VMEM),
            pl.BlockSpec(memory_space=pltpu.MemorySpace.VMEM),
            pl.BlockSpec(memory_space=pltpu.MemorySpace.VMEM),
            pl.BlockSpec(memory_space=pltpu.MemorySpace.VMEM),
            pl.BlockSpec(memory_space=pltpu.MemorySpace.HBM),
        ],
        out_specs=pl.BlockSpec(memory_space=pltpu.MemorySpace.HBM),
        out_shape=jax.ShapeDtypeStruct((N, F), jnp.float32),
        scratch_shapes=[
            pltpu.VMEM((N, F), jnp.float32),
            pltpu.SemaphoreType.DMA((NBLK,)),
            pltpu.SemaphoreType.DMA((NBLK,)),
        ],
    )(b2, b3, part, W0, b0.reshape(1, H), W1, b1.reshape(1, F), x)

    return out


# speedup vs baseline: 3.6010x; 3.6010x over previous
import jax
import jax.numpy as jnp
from jax import lax
from jax.experimental import pallas as pl
from jax.experimental.pallas import tpu as pltpu

N = 16384
F = 512
H = 128
S = 8
BLK = 512
NBLK = N // BLK


def _body(b2_ref, b3_ref, W0_ref, b0_ref, W1_ref, b1_ref, x_hbm, out_hbm,
          xbuf_ref, rsem, wsem):
    read_copies = []
    for k in range(NBLK):
        c = pltpu.make_async_copy(
            x_hbm.at[pl.ds(k * BLK, BLK), :],
            xbuf_ref.at[pl.ds(k * BLK, BLK), :],
            rsem.at[k])
        c.start()
        read_copies.append(c)

    b2 = b2_ref[...]
    cnt = jnp.concatenate(
        [jnp.sum((b2 == s).astype(jnp.float32))[None] for s in range(S)])
    inv_cnt = 1.0 / jnp.maximum(cnt, 1.0)[:, None]
    W0 = W0_ref[...]
    b0 = b0_ref[...]
    W1 = W1_ref[...]
    b1 = b1_ref[...]

    def onehot(k):
        ids = b3_ref[k, 0, :]
        return (ids[:, None] == lax.broadcasted_iota(jnp.int32, (BLK, S), 1)
                ).astype(jnp.float32)

    acc = jnp.zeros((S, F), jnp.float32)
    for k in range(NBLK):
        read_copies[k].wait()
        xi = xbuf_ref[pl.ds(k * BLK, BLK), :]
        acc = acc + lax.dot_general(onehot(k), xi, (((0,), (0,)), ((), ())),
                                    preferred_element_type=jnp.float32)

    mean = acc * inv_cnt
    h = jnp.maximum(
        lax.dot_general(mean, W0, (((1,), (0,)), ((), ())),
                        preferred_element_type=jnp.float32) + b0, 0.0)
    z = lax.dot_general(h, W1, (((1,), (0,)), ((), ())),
                        preferred_element_type=jnp.float32) + b1
    gate = 1.0 / (1.0 + jnp.exp(-z))

    write_copies = []
    for k in range(NBLK):
        y = lax.dot_general(onehot(k), gate, (((1,), (0,)), ((), ())),
                            preferred_element_type=jnp.float32)
        xbuf_ref[pl.ds(k * BLK, BLK), :] *= y
        c = pltpu.make_async_copy(
            xbuf_ref.at[pl.ds(k * BLK, BLK), :],
            out_hbm.at[pl.ds(k * BLK, BLK), :],
            wsem.at[k])
        c.start()
        write_copies.append(c)

    for c in write_copies:
        c.wait()


def kernel(x, batch, W0, b0, W1, b1):
    batch32 = batch.astype(jnp.int32)
    b3 = batch32.reshape(NBLK, 1, BLK)
    b2 = batch32.reshape(128, 128)

    out = pl.pallas_call(
        _body,
        in_specs=[
            pl.BlockSpec(memory_space=pltpu.MemorySpace.VMEM),
            pl.BlockSpec(memory_space=pltpu.MemorySpace.VMEM),
            pl.BlockSpec(memory_space=pltpu.MemorySpace.VMEM),
            pl.BlockSpec(memory_space=pltpu.MemorySpace.VMEM),
            pl.BlockSpec(memory_space=pltpu.MemorySpace.VMEM),
            pl.BlockSpec(memory_space=pltpu.MemorySpace.VMEM),
            pl.BlockSpec(memory_space=pltpu.MemorySpace.HBM),
        ],
        out_specs=pl.BlockSpec(memory_space=pltpu.MemorySpace.HBM),
        out_shape=jax.ShapeDtypeStruct((N, F), jnp.float32),
        scratch_shapes=[
            pltpu.VMEM((N, F), jnp.float32),
            pltpu.SemaphoreType.DMA((NBLK,)),
            pltpu.SemaphoreType.DMA((NBLK,)),
        ],
    )(b2, b3, W0, b0.reshape(1, H), W1, b1.reshape(1, F), x)

    return out
